# baseline (speedup 1.0000x reference)
import functools

import numpy as np

import jax
import jax.numpy as jnp
from jax import lax
from jax.experimental import pallas as pl
from jax.experimental.pallas import tpu as pltpu
from jax.experimental.pallas import tpu_sc as plsc

N_U = 50000
N_I = 50000
N = N_U + N_I
E = 1600000
D = 32
LAYERS = 3

NC = 2
NS = 16
HD = D // NC
CHUNK = 128
SBLK = 16
NBUF = 4
EROWS = 12544
ROWS_PER_TILE = EROWS // NS
NBLOCKS = ROWS_PER_TILE // SBLK
N3 = 100352

WB3 = N3 // NS
ZCH = WB3 // CHUNK

_DNUMS = lax.GatherDimensionNumbers(
    offset_dims=(), collapsed_slice_dims=(0,), start_index_map=(0,))


def _layer_body(lo, hi, srcr, dstr, valsr, out_lo, out_hi,
                src_blk, dst_blk, vals_blk,
                sidx0, sidx1, sidx2, sidx3,
                rows0, rows1, rows2, rows3,
                sg0, sg1, sg2, sg3, ss0, ss1, ss2, ss3, acc):
    c = lax.axis_index("c")
    s = lax.axis_index("s")

    rows = (rows0, rows1, rows2, rows3)
    sidx = (sidx0, sidx1, sidx2, sidx3)
    sem_g = (sg0, sg1, sg2, sg3)
    sem_s = (ss0, ss1, ss2, ss3)

    zero = jnp.zeros((16,), jnp.float32)
    izero = jnp.zeros((16,), jnp.int32)
    for buf in rows:
        for e in range(CHUNK):
            buf[e, pl.ds(0, HD)] = zero
    for ix in sidx:
        for k in range(CHUNK // 16):
            ix[pl.ds(k * 16, 16)] = izero

    def _zero(k, _):
        pltpu.sync_copy(rows0, acc.at[pl.ds(s * WB3 + k * CHUNK, CHUNK)])
        return ()
    lax.fori_loop(0, ZCH, _zero, ())
    plsc.subcore_barrier()

    for p in range(NBUF):
        pltpu.async_copy(rows[p], acc.at[sidx[p]], sem_s[p], add=True)

    def _scatter_done(p):
        pltpu.make_async_copy(rows[p], acc.at[sidx[p]], sem_s[p]).wait()

    def _gather(jrow, q):
        @pl.when(c == 0)
        def _g0():
            pltpu.async_copy(lo.at[src_blk.at[jrow]], rows[q], sem_g[q])

        @pl.when(c == 1)
        def _g1():
            pltpu.async_copy(hi.at[src_blk.at[jrow]], rows[q], sem_g[q])

    def _block(b, _):
        row0 = s * ROWS_PER_TILE + b * SBLK
        pltpu.sync_copy(srcr.at[pl.ds(row0, SBLK)], src_blk)
        pltpu.sync_copy(dstr.at[pl.ds(row0, SBLK)], dst_blk)
        pltpu.sync_copy(valsr.at[pl.ds(row0, SBLK)], vals_blk)

        _scatter_done(0)
        _gather(0, 0)
        _scatter_done(1)
        _gather(1, 1)

        for j in range(SBLK):
            p = j % NBUF
            if j + 2 < SBLK:
                q = (j + 2) % NBUF
                _scatter_done(q)
                _gather(j + 2, q)
            for k in range(CHUNK // 16):
                sidx[p][pl.ds(k * 16, 16)] = dst_blk[j, pl.ds(k * 16, 16)]
            pltpu.make_async_copy(lo.at[src_blk.at[j]], rows[p], sem_g[p]).wait()

            for g in range(CHUNK // 16):
                sv = vals_blk[j, pl.ds(g * 16, 16)]
                for i in range(16):
                    vv = lax.gather(
                        sv, jnp.full((16, 1), i, jnp.int32), _DNUMS,
                        slice_sizes=(1,),
                        mode=lax.GatherScatterMode.PROMISE_IN_BOUNDS)
                    e = g * 16 + i
                    rows[p][e, pl.ds(0, HD)] = rows[p][e, pl.ds(0, HD)] * vv

            pltpu.async_copy(rows[p], acc.at[sidx[p]], sem_s[p], add=True)
        return ()
    lax.fori_loop(0, NBLOCKS, _block, ())

    for p in range(NBUF):
        _scatter_done(p)
    plsc.subcore_barrier()

    @pl.when(c == 0)
    def _wb0():
        pltpu.sync_copy(acc.at[pl.ds(s * WB3, WB3)],
                        out_lo.at[pl.ds(s * WB3, WB3)])

    @pl.when(c == 1)
    def _wb1():
        pltpu.sync_copy(acc.at[pl.ds(s * WB3, WB3)],
                        out_hi.at[pl.ds(s * WB3, WB3)])


_layer = functools.partial(
    pl.kernel,
    out_type=(jax.ShapeDtypeStruct((N3, HD), jnp.float32),
              jax.ShapeDtypeStruct((N3, HD), jnp.float32)),
    mesh=plsc.VectorSubcoreMesh(core_axis_name="c", subcore_axis_name="s"),
    compiler_params=pltpu.CompilerParams(use_tc_tiling_on_sc=False,
                                         needs_layout_passes=False),
    scratch_types=[
        pltpu.VMEM((SBLK, CHUNK), jnp.int32),
        pltpu.VMEM((SBLK, CHUNK), jnp.int32),
        pltpu.VMEM((SBLK, CHUNK), jnp.float32),
        pltpu.VMEM((CHUNK,), jnp.int32),
        pltpu.VMEM((CHUNK,), jnp.int32),
        pltpu.VMEM((CHUNK,), jnp.int32),
        pltpu.VMEM((CHUNK,), jnp.int32),
        pltpu.VMEM((CHUNK, HD), jnp.float32),
        pltpu.VMEM((CHUNK, HD), jnp.float32),
        pltpu.VMEM((CHUNK, HD), jnp.float32),
        pltpu.VMEM((CHUNK, HD), jnp.float32),
        pltpu.SemaphoreType.DMA,
        pltpu.SemaphoreType.DMA,
        pltpu.SemaphoreType.DMA,
        pltpu.SemaphoreType.DMA,
        pltpu.SemaphoreType.DMA,
        pltpu.SemaphoreType.DMA,
        pltpu.SemaphoreType.DMA,
        pltpu.SemaphoreType.DMA,
        pltpu.VMEM_SHARED((N3, HD), jnp.float32),
    ],
)(_layer_body)


def _mean3_body(a, b, c, o):
    o[...] = (a[...] + b[...] + c[...]) * jnp.float32(1.0 / 3.0)


def _mean3h(e1, e2, e3):
    flat = (12544, 128)
    spec = pl.BlockSpec((784, 128), lambda i: (i, 0))
    out = pl.pallas_call(
        _mean3_body,
        out_shape=jax.ShapeDtypeStruct(flat, jnp.float32),
        grid=(16,),
        in_specs=[spec, spec, spec],
        out_specs=spec,
    )(e1.reshape(flat), e2.reshape(flat), e3.reshape(flat))
    return out.reshape(N3, HD)


@jax.jit
def kernel(user_emb, item_emb, adj_indices, adj_values):
    dst = adj_indices[0].astype(jnp.int32)
    src = adj_indices[1].astype(jnp.int32)
    vals = adj_values.astype(jnp.float32)

    zpad = jnp.zeros((N3 - N, HD), jnp.float32)
    lo = jnp.concatenate([user_emb[:, :HD], item_emb[:, :HD], zpad], axis=0)
    hi = jnp.concatenate([user_emb[:, HD:], item_emb[:, HD:], zpad], axis=0)

    pad = EROWS * CHUNK - E
    src2d = jnp.concatenate([src, jnp.zeros((pad,), jnp.int32)]).reshape(EROWS, CHUNK)
    dst2d = jnp.concatenate([dst, jnp.zeros((pad,), jnp.int32)]).reshape(EROWS, CHUNK)
    vals2d = jnp.concatenate([vals, jnp.zeros((pad,), jnp.float32)]).reshape(EROWS, CHUNK)

    los, his = [], []
    for _ in range(LAYERS):
        lo, hi = _layer(lo, hi, src2d, dst2d, vals2d)
        los.append(lo)
        his.append(hi)

    mlo = _mean3h(*los)
    mhi = _mean3h(*his)
    full = jnp.concatenate([mlo[:N], mhi[:N]], axis=1)
    return (full[:N_U], full[N_U:])
